# baseline (speedup 1.0000x reference)
import numpy as np
import jax
import jax.numpy as jnp
from jax.experimental import pallas as pl

B, L, NV = 16, 512, 128
D, PL_, ST = 128, 16, 8
NH = 8
DFF = 256
PRED = 96
P = (L + ST - PL_) // ST + 1
DH = D // NH
BN = B * NV
NLITE = 12
QB = 512
F32 = jnp.float32


def _np_pos_emb(n, d):
    pos = np.arange(n)[:, None].astype(np.float32)
    div = np.exp(np.arange(0, d, 2).astype(np.float32) * -(np.log(10000.0) / d))
    pe = np.zeros((n, d), np.float32)
    pe[:, 0::2] = np.sin(pos * div)
    pe[:, 1::2] = np.cos(pos * div)
    return pe


_POS = _np_pos_emb(P, D)

_A = np.zeros((NLITE, PL_, PRED), np.float32)
for _i in range(NLITE):
    for _j in range(PL_):
        _t = 8 * _i + _j - 8
        if 0 <= _t < PRED:
            _A[_i, _j, _t] = 1.0


_SQRT_HALF = 0.7071067811865476


def _gelu(x):
    return 0.5 * x * (1.0 + jax.lax.erf(x * _SQRT_HALF))


def _ln(x, g, b, eps=1e-5):
    m = jnp.mean(x, axis=-1, keepdims=True)
    v = jnp.mean((x - m) ** 2, axis=-1, keepdims=True)
    return (x - m) / jnp.sqrt(v + eps) * g + b


def _k1_body(x_ref, raw_ref, wval_ref, pos_ref,
             ws1_ref, bs1_ref, gs_ref, bsln_ref, ws2_ref, bs2_ref,
             wq_ref, bq_ref, wk_ref, bk_ref, wv_ref, bv_ref,
             wl1_ref, bl1_ref, wl2_ref, bl2_ref, wr_ref, br_ref, a_ref,
             sel_ref, q_ref, k_ref, v_ref, oh_ref, lite_ref, ms_ref):
    xb = x_ref[0]
    m = jnp.mean(xb, axis=0)
    c = xb - m[None, :]
    var = jnp.mean(c * c, axis=0)
    s = jnp.sqrt(var + 1e-5)
    mc = jnp.transpose(m[None, :])
    sc = jnp.transpose(s[None, :])

    patches = raw_ref[0]
    pn = (patches - mc[:, :, None]) / sc[:, :, None]
    emb2 = jnp.dot(pn.reshape(NV * P, PL_), wval_ref[...],
                   preferred_element_type=F32)
    emb3 = emb2.reshape(NV, P, D) + pos_ref[...][None]

    e2 = emb3.reshape(NV * P, D)
    h = _gelu(jnp.dot(e2, ws1_ref[...], preferred_element_type=F32)
              + bs1_ref[...])
    jm = jnp.full((64, 64), 1.0 / 64, F32)
    mh = jnp.dot(h, jm, preferred_element_type=F32,
                 precision=jax.lax.Precision.HIGHEST)
    ch = h - mh
    vh = jnp.dot(ch * ch, jm, preferred_element_type=F32,
                 precision=jax.lax.Precision.HIGHEST)
    hn = ch / jnp.sqrt(vh + 1e-5) * gs_ref[...] + bsln_ref[...]
    hb = hn.astype(jnp.bfloat16).astype(F32)
    wb = ws2_ref[...].astype(jnp.bfloat16).astype(F32)
    scores = (jnp.sum(hb.reshape(NV, P, 64) * wb[None], axis=-1)
              + bs2_ref[0, 0])

    amax = jnp.argmax(scores, axis=-1)
    iota_p = jax.lax.broadcasted_iota(jnp.int32, (NV, P), 1)
    msel = (iota_p == amax[:, None]).astype(F32)

    sel = jnp.sum(emb3 * msel[:, :, None], axis=1)
    sel_ref[0] = sel
    q_ref[0] = jnp.dot(sel, wq_ref[...], preferred_element_type=F32) + bq_ref[...]
    k_ref[0] = jnp.dot(sel, wk_ref[...], preferred_element_type=F32) + bk_ref[...]
    v_ref[0] = jnp.dot(sel, wv_ref[...], preferred_element_type=F32) + bv_ref[...]

    oh = msel[:, P - NLITE:]
    oh_ref[0] = oh

    es = emb3[:, P - NLITE:, :].reshape(NV * NLITE, D)
    hl = _gelu(jnp.dot(es, wl1_ref[...], preferred_element_type=F32)
               + bl1_ref[...])
    lf = jnp.dot(hl, wl2_ref[...], preferred_element_type=F32) + bl2_ref[...]
    rbl = jnp.dot(lf, wr_ref[...], preferred_element_type=F32) + br_ref[...]
    rb3 = rbl.reshape(NV, NLITE, PL_) * (1.0 - oh)[:, :, None]
    acc = jnp.zeros((NV, PRED), F32)
    for i in range(NLITE):
        acc = acc + jnp.dot(rb3[:, i, :], a_ref[i], preferred_element_type=F32)
    lite_ref[0] = acc
    ms_ref[0] = jnp.concatenate([mc, sc], axis=1)


def _k2_body(q_ref, k_ref, v_ref, o_ref):
    q = q_ref[0]
    k = k_ref[0]
    v = v_ref[0]
    sc = jax.lax.dot_general(q, k, (((1,), (1,)), ((), ())),
                             preferred_element_type=F32) * (1.0 / (DH ** 0.5))
    mx = jnp.max(sc, axis=-1, keepdims=True)
    e = jnp.exp(sc - mx)
    p_attn = e / jnp.sum(e, axis=-1, keepdims=True)
    o_ref[0] = jnp.dot(p_attn, v, preferred_element_type=F32)


def _k3_body(sel_ref, attn_ref, wo_ref, bo_ref, wf1_ref, bf1_ref,
             wf2_ref, bf2_ref, g1_ref, b1_ref, g2_ref, b2_ref,
             ge_ref, be_ref, wr_ref, br_ref, a_ref,
             oh_ref, lite_ref, ms_ref, out_ref):
    x0 = sel_ref[0] + (jnp.dot(attn_ref[0], wo_ref[...],
                               preferred_element_type=F32) + bo_ref[...])
    x1 = _ln(x0, g1_ref[...], b1_ref[...])
    y = (jnp.dot(_gelu(jnp.dot(x1, wf1_ref[...], preferred_element_type=F32)
                       + bf1_ref[...]),
                 wf2_ref[...], preferred_element_type=F32) + bf2_ref[...])
    x2 = _ln(x1 + y, g2_ref[...], b2_ref[...])
    f = _ln(x2, ge_ref[...], be_ref[...])
    frb = jnp.dot(f, wr_ref[...], preferred_element_type=F32) + br_ref[...]
    oh = oh_ref[0]
    acc = lite_ref[0]
    for i in range(NLITE):
        acc = acc + jnp.dot(frb * oh[:, i][:, None], a_ref[i],
                            preferred_element_type=F32)
    msb = ms_ref[0]
    y96 = acc * msb[:, 1:2] + msb[:, 0:1]
    out_ref[0] = jnp.transpose(y96)


def kernel(x_enc, params):
    p = params
    xv = jnp.transpose(x_enc, (0, 2, 1))
    xp = jnp.concatenate([xv, jnp.repeat(xv[..., -1:], ST, axis=-1)], axis=-1)
    uidx = np.arange(P)[:, None] * ST + np.arange(PL_)[None, :]
    raw = xp[..., uidx]

    pos = jnp.asarray(_POS)
    amat = jnp.asarray(_A)
    r1 = lambda a: a.reshape(1, -1)

    full = lambda shp: pl.BlockSpec(shp, lambda b: (0,) * len(shp))

    k1_out = pl.pallas_call(
        _k1_body,
        grid=(B,),
        in_specs=[
            pl.BlockSpec((1, L, NV), lambda b: (b, 0, 0)),
            pl.BlockSpec((1, NV, P, PL_), lambda b: (b, 0, 0, 0)),
            full((PL_, D)), full((P, D)),
            full((D, 64)), full((1, 64)), full((1, 64)), full((1, 64)),
            full((1, 64)), full((1, 1)),
            full((D, D)), full((1, D)),
            full((D, D)), full((1, D)),
            full((D, D)), full((1, D)),
            full((D, 64)), full((1, 64)), full((64, D)), full((1, D)),
            full((D, PL_)), full((1, PL_)), full((NLITE, PL_, PRED)),
        ],
        out_specs=[
            pl.BlockSpec((1, NV, D), lambda b: (b, 0, 0)),
            pl.BlockSpec((1, NV, D), lambda b: (b, 0, 0)),
            pl.BlockSpec((1, NV, D), lambda b: (b, 0, 0)),
            pl.BlockSpec((1, NV, D), lambda b: (b, 0, 0)),
            pl.BlockSpec((1, NV, NLITE), lambda b: (b, 0, 0)),
            pl.BlockSpec((1, NV, PRED), lambda b: (b, 0, 0)),
            pl.BlockSpec((1, NV, 2), lambda b: (b, 0, 0)),
        ],
        out_shape=[
            jax.ShapeDtypeStruct((B, NV, D), F32),
            jax.ShapeDtypeStruct((B, NV, D), F32),
            jax.ShapeDtypeStruct((B, NV, D), F32),
            jax.ShapeDtypeStruct((B, NV, D), F32),
            jax.ShapeDtypeStruct((B, NV, NLITE), F32),
            jax.ShapeDtypeStruct((B, NV, PRED), F32),
            jax.ShapeDtypeStruct((B, NV, 2), F32),
        ],
    )(x_enc, raw, p["W_val"], pos,
      p["Ws1"], r1(p["bs1"]), r1(p["gs"]), r1(p["bs_ln"]),
      r1(p["Ws2"][:, 0]), p["bs2"].reshape(1, 1),
      p["Wq"], r1(p["bq"]), p["Wk"], r1(p["bk"]), p["Wv"], r1(p["bv"]),
      p["Wl1"], r1(p["bl1"]), p["Wl2"], r1(p["bl2"]),
      p["Wr"], r1(p["br"]), amat)

    sel_emb, qf, kf, vf, oh12, lite96, ms = k1_out

    qh = qf.reshape(BN, NH, DH).transpose(1, 0, 2)
    kh = kf.reshape(BN, NH, DH).transpose(1, 0, 2)
    vh = vf.reshape(BN, NH, DH).transpose(1, 0, 2)

    attn = pl.pallas_call(
        _k2_body,
        grid=(NH, BN // QB),
        in_specs=[
            pl.BlockSpec((1, QB, DH), lambda h, qb: (h, qb, 0)),
            pl.BlockSpec((1, BN, DH), lambda h, qb: (h, 0, 0)),
            pl.BlockSpec((1, BN, DH), lambda h, qb: (h, 0, 0)),
        ],
        out_specs=pl.BlockSpec((1, QB, DH), lambda h, qb: (h, qb, 0)),
        out_shape=jax.ShapeDtypeStruct((NH, BN, DH), F32),
    )(qh, kh, vh)

    attn_t = attn.transpose(1, 0, 2).reshape(B, NV, D)

    out = pl.pallas_call(
        _k3_body,
        grid=(B,),
        in_specs=[
            pl.BlockSpec((1, NV, D), lambda b: (b, 0, 0)),
            pl.BlockSpec((1, NV, D), lambda b: (b, 0, 0)),
            full((D, D)), full((1, D)),
            full((D, DFF)), full((1, DFF)), full((DFF, D)), full((1, D)),
            full((1, D)), full((1, D)), full((1, D)), full((1, D)),
            full((1, D)), full((1, D)),
            full((D, PL_)), full((1, PL_)), full((NLITE, PL_, PRED)),
            pl.BlockSpec((1, NV, NLITE), lambda b: (b, 0, 0)),
            pl.BlockSpec((1, NV, PRED), lambda b: (b, 0, 0)),
            pl.BlockSpec((1, NV, 2), lambda b: (b, 0, 0)),
        ],
        out_specs=pl.BlockSpec((1, PRED, NV), lambda b: (b, 0, 0)),
        out_shape=jax.ShapeDtypeStruct((B, PRED, NV), F32),
    )(sel_emb, attn_t,
      p["Wo"], r1(p["bo"]), p["Wf1"], r1(p["bf1"]), p["Wf2"], r1(p["bf2"]),
      r1(p["g1"]), r1(p["b1"]), r1(p["g2"]), r1(p["b2"]),
      r1(p["gE"]), r1(p["bE"]),
      p["Wr"], r1(p["br"]), amat, oh12, lite96, ms)

    return out


# speedup vs baseline: 1.5281x; 1.5281x over previous
import numpy as np
import jax
import jax.numpy as jnp
from jax.experimental import pallas as pl

B, L, NV = 16, 512, 128
D, PL_, ST = 128, 16, 8
NH = 8
DFF = 256
PRED = 96
P = (L + ST - PL_) // ST + 1
DH = D // NH
BN = B * NV
NLITE = 12
QB = 512
F32 = jnp.float32


def _np_pos_emb(n, d):
    pos = np.arange(n)[:, None].astype(np.float32)
    div = np.exp(np.arange(0, d, 2).astype(np.float32) * -(np.log(10000.0) / d))
    pe = np.zeros((n, d), np.float32)
    pe[:, 0::2] = np.sin(pos * div)
    pe[:, 1::2] = np.cos(pos * div)
    return pe


_POS = _np_pos_emb(P, D)

_A = np.zeros((NLITE, PL_, PRED), np.float32)
for _i in range(NLITE):
    for _j in range(PL_):
        _t = 8 * _i + _j - 8
        if 0 <= _t < PRED:
            _A[_i, _j, _t] = 1.0


_SQRT_HALF = 0.7071067811865476


def _gelu(x):
    return 0.5 * x * (1.0 + jax.lax.erf(x * _SQRT_HALF))


def _ln(x, g, b, eps=1e-5):
    m = jnp.mean(x, axis=-1, keepdims=True)
    v = jnp.mean((x - m) ** 2, axis=-1, keepdims=True)
    return (x - m) / jnp.sqrt(v + eps) * g + b


def _k1_body(x_ref, raw_ref, wval_ref, pos_ref,
             ws1_ref, bs1_ref, gs_ref, bsln_ref, ws2_ref, bs2_ref,
             wq_ref, bq_ref, wk_ref, bk_ref, wv_ref, bv_ref,
             wl1_ref, bl1_ref, wl2_ref, bl2_ref, wr_ref, br_ref, a_ref,
             sel_ref, q_ref, k_ref, v_ref, oh_ref, lite_ref, ms_ref):
    xb = x_ref[0]
    m = jnp.mean(xb, axis=0)
    c = xb - m[None, :]
    var = jnp.mean(c * c, axis=0)
    s = jnp.sqrt(var + 1e-5)
    mc = jnp.transpose(m[None, :])
    sc = jnp.transpose(s[None, :])

    patches = raw_ref[0]
    pn = (patches - mc[:, :, None]) / sc[:, :, None]
    emb2 = jnp.dot(pn.reshape(NV * P, PL_), wval_ref[...],
                   preferred_element_type=F32)
    emb3 = emb2.reshape(NV, P, D) + pos_ref[...][None]

    e2 = emb3.reshape(NV * P, D)
    h = _gelu(jnp.dot(e2, ws1_ref[...], preferred_element_type=F32)
              + bs1_ref[...])
    hn = _ln(h, gs_ref[...], bsln_ref[...])
    hb = hn.astype(jnp.bfloat16).astype(F32)
    wb = ws2_ref[...].astype(jnp.bfloat16).astype(F32)
    scores = (jnp.sum(hb.reshape(NV, P, 64) * wb[None], axis=-1)
              + bs2_ref[0, 0])

    amax = jnp.argmax(scores, axis=-1)
    iota_p = jax.lax.broadcasted_iota(jnp.int32, (NV, P), 1)
    msel = (iota_p == amax[:, None]).astype(F32)

    sel = jnp.sum(emb3 * msel[:, :, None], axis=1)
    sel_ref[0] = sel
    q_ref[0] = jnp.dot(sel, wq_ref[...], preferred_element_type=F32) + bq_ref[...]
    k_ref[0] = jnp.dot(sel, wk_ref[...], preferred_element_type=F32) + bk_ref[...]
    v_ref[0] = jnp.dot(sel, wv_ref[...], preferred_element_type=F32) + bv_ref[...]

    oh = msel[:, P - NLITE:]
    oh_ref[0] = oh

    es = emb3[:, P - NLITE:, :].reshape(NV * NLITE, D)
    hl = _gelu(jnp.dot(es, wl1_ref[...], preferred_element_type=F32)
               + bl1_ref[...])
    lf = jnp.dot(hl, wl2_ref[...], preferred_element_type=F32) + bl2_ref[...]
    rbl = jnp.dot(lf, wr_ref[...], preferred_element_type=F32) + br_ref[...]
    rb3 = rbl.reshape(NV, NLITE, PL_) * (1.0 - oh)[:, :, None]
    acc = jnp.zeros((NV, PRED), F32)
    for i in range(NLITE):
        acc = acc + jnp.dot(rb3[:, i, :], a_ref[i], preferred_element_type=F32)
    lite_ref[0] = acc
    ms_ref[0] = jnp.concatenate([mc, sc], axis=1)


def _k2_body(q_ref, k_ref, v_ref, o_ref):
    q = q_ref[0]
    k = k_ref[0]
    v = v_ref[0]
    sc = jax.lax.dot_general(q, k, (((1,), (1,)), ((), ())),
                             preferred_element_type=F32) * (1.0 / (DH ** 0.5))
    mx = jnp.max(sc, axis=-1, keepdims=True)
    e = jnp.exp(sc - mx)
    p_attn = e / jnp.sum(e, axis=-1, keepdims=True)
    o_ref[0] = jnp.dot(p_attn, v, preferred_element_type=F32)


def _k3_body(sel_ref, attn_ref, wo_ref, bo_ref, wf1_ref, bf1_ref,
             wf2_ref, bf2_ref, g1_ref, b1_ref, g2_ref, b2_ref,
             ge_ref, be_ref, wr_ref, br_ref, a_ref,
             oh_ref, lite_ref, ms_ref, out_ref):
    x0 = sel_ref[0] + (jnp.dot(attn_ref[0], wo_ref[...],
                               preferred_element_type=F32) + bo_ref[...])
    x1 = _ln(x0, g1_ref[...], b1_ref[...])
    y = (jnp.dot(_gelu(jnp.dot(x1, wf1_ref[...], preferred_element_type=F32)
                       + bf1_ref[...]),
                 wf2_ref[...], preferred_element_type=F32) + bf2_ref[...])
    x2 = _ln(x1 + y, g2_ref[...], b2_ref[...])
    f = _ln(x2, ge_ref[...], be_ref[...])
    frb = jnp.dot(f, wr_ref[...], preferred_element_type=F32) + br_ref[...]
    oh = oh_ref[0]
    acc = lite_ref[0]
    for i in range(NLITE):
        acc = acc + jnp.dot(frb * oh[:, i][:, None], a_ref[i],
                            preferred_element_type=F32)
    msb = ms_ref[0]
    y96 = acc * msb[:, 1:2] + msb[:, 0:1]
    out_ref[0] = jnp.transpose(y96)


def kernel(x_enc, params):
    p = params
    xv = jnp.transpose(x_enc, (0, 2, 1))
    xp = jnp.concatenate([xv, jnp.repeat(xv[..., -1:], ST, axis=-1)], axis=-1)
    uidx = np.arange(P)[:, None] * ST + np.arange(PL_)[None, :]
    raw = xp[..., uidx]

    pos = jnp.asarray(_POS)
    amat = jnp.asarray(_A)
    r1 = lambda a: a.reshape(1, -1)

    full = lambda shp: pl.BlockSpec(shp, lambda b: (0,) * len(shp))

    k1_out = pl.pallas_call(
        _k1_body,
        grid=(B,),
        in_specs=[
            pl.BlockSpec((1, L, NV), lambda b: (b, 0, 0)),
            pl.BlockSpec((1, NV, P, PL_), lambda b: (b, 0, 0, 0)),
            full((PL_, D)), full((P, D)),
            full((D, 64)), full((1, 64)), full((1, 64)), full((1, 64)),
            full((1, 64)), full((1, 1)),
            full((D, D)), full((1, D)),
            full((D, D)), full((1, D)),
            full((D, D)), full((1, D)),
            full((D, 64)), full((1, 64)), full((64, D)), full((1, D)),
            full((D, PL_)), full((1, PL_)), full((NLITE, PL_, PRED)),
        ],
        out_specs=[
            pl.BlockSpec((1, NV, D), lambda b: (b, 0, 0)),
            pl.BlockSpec((1, NV, D), lambda b: (b, 0, 0)),
            pl.BlockSpec((1, NV, D), lambda b: (b, 0, 0)),
            pl.BlockSpec((1, NV, D), lambda b: (b, 0, 0)),
            pl.BlockSpec((1, NV, NLITE), lambda b: (b, 0, 0)),
            pl.BlockSpec((1, NV, PRED), lambda b: (b, 0, 0)),
            pl.BlockSpec((1, NV, 2), lambda b: (b, 0, 0)),
        ],
        out_shape=[
            jax.ShapeDtypeStruct((B, NV, D), F32),
            jax.ShapeDtypeStruct((B, NV, D), F32),
            jax.ShapeDtypeStruct((B, NV, D), F32),
            jax.ShapeDtypeStruct((B, NV, D), F32),
            jax.ShapeDtypeStruct((B, NV, NLITE), F32),
            jax.ShapeDtypeStruct((B, NV, PRED), F32),
            jax.ShapeDtypeStruct((B, NV, 2), F32),
        ],
    )(x_enc, raw, p["W_val"], pos,
      p["Ws1"], r1(p["bs1"]), r1(p["gs"]), r1(p["bs_ln"]),
      r1(p["Ws2"][:, 0]), p["bs2"].reshape(1, 1),
      p["Wq"], r1(p["bq"]), p["Wk"], r1(p["bk"]), p["Wv"], r1(p["bv"]),
      p["Wl1"], r1(p["bl1"]), p["Wl2"], r1(p["bl2"]),
      p["Wr"], r1(p["br"]), amat)

    sel_emb, qf, kf, vf, oh12, lite96, ms = k1_out

    qh = qf.reshape(BN, NH, DH).transpose(1, 0, 2)
    kh = kf.reshape(BN, NH, DH).transpose(1, 0, 2)
    vh = vf.reshape(BN, NH, DH).transpose(1, 0, 2)

    attn = pl.pallas_call(
        _k2_body,
        grid=(NH, BN // QB),
        in_specs=[
            pl.BlockSpec((1, QB, DH), lambda h, qb: (h, qb, 0)),
            pl.BlockSpec((1, BN, DH), lambda h, qb: (h, 0, 0)),
            pl.BlockSpec((1, BN, DH), lambda h, qb: (h, 0, 0)),
        ],
        out_specs=pl.BlockSpec((1, QB, DH), lambda h, qb: (h, qb, 0)),
        out_shape=jax.ShapeDtypeStruct((NH, BN, DH), F32),
    )(qh, kh, vh)

    attn_t = attn.transpose(1, 0, 2).reshape(B, NV, D)

    out = pl.pallas_call(
        _k3_body,
        grid=(B,),
        in_specs=[
            pl.BlockSpec((1, NV, D), lambda b: (b, 0, 0)),
            pl.BlockSpec((1, NV, D), lambda b: (b, 0, 0)),
            full((D, D)), full((1, D)),
            full((D, DFF)), full((1, DFF)), full((DFF, D)), full((1, D)),
            full((1, D)), full((1, D)), full((1, D)), full((1, D)),
            full((1, D)), full((1, D)),
            full((D, PL_)), full((1, PL_)), full((NLITE, PL_, PRED)),
            pl.BlockSpec((1, NV, NLITE), lambda b: (b, 0, 0)),
            pl.BlockSpec((1, NV, PRED), lambda b: (b, 0, 0)),
            pl.BlockSpec((1, NV, 2), lambda b: (b, 0, 0)),
        ],
        out_specs=pl.BlockSpec((1, PRED, NV), lambda b: (b, 0, 0)),
        out_shape=jax.ShapeDtypeStruct((B, PRED, NV), F32),
    )(sel_emb, attn_t,
      p["Wo"], r1(p["bo"]), p["Wf1"], r1(p["bf1"]), p["Wf2"], r1(p["bf2"]),
      r1(p["g1"]), r1(p["b1"]), r1(p["g2"]), r1(p["b2"]),
      r1(p["gE"]), r1(p["bE"]),
      p["Wr"], r1(p["br"]), amat, oh12, lite96, ms)

    return out
